# Initial kernel scaffold; baseline (speedup 1.0000x reference)
import functools

import jax
import jax.numpy as jnp
from jax import lax
from jax.experimental import pallas as pl
from jax.experimental.pallas import tpu as pltpu
from jax.experimental.pallas import tpu_sc as plsc

N = 10000
E = 320000
D_IN = 128
DH = 64
NG = 64

NC = 2
NS = 16
NW = NC * NS
EPW = E // NW
CH = 80
NCH = EPW // CH
RPT = N // NS
ZR = 125

_sc_mesh = plsc.VectorSubcoreMesh(core_axis_name="c", subcore_axis_name="s")


@functools.partial(
    pl.kernel,
    mesh=_sc_mesh,
    out_type=jax.ShapeDtypeStruct((NC, N, DH), jnp.float32),
    scratch_types=[
        pltpu.VMEM((CH,), jnp.int32),
        pltpu.VMEM((CH,), jnp.int32),
        pltpu.VMEM((CH, DH), jnp.float32),
        pltpu.VMEM((ZR, DH), jnp.float32),
        pltpu.VMEM_SHARED((N, DH), jnp.float32),
        pltpu.SemaphoreType.DMA,
    ],
)
def _edge_prop(hs_hbm, src_hbm, dst_hbm, out_hbm, src_v, dst_v, rows_v,
               zer_v, acc_sh, sem):
    c = lax.axis_index("c")
    s = lax.axis_index("s")
    wid = c * NS + s

    def zfill(i, _):
        def zfill_lane(j, _):
            zer_v[i, pl.ds(j * 16, 16)] = jnp.zeros((16,), jnp.float32)
            return 0
        return lax.fori_loop(0, DH // 16, zfill_lane, 0)
    lax.fori_loop(0, ZR, zfill, 0)

    def zcopy(k, _):
        pltpu.sync_copy(zer_v, acc_sh.at[pl.ds(s * RPT + k * ZR, ZR)])
        return 0
    lax.fori_loop(0, RPT // ZR, zcopy, 0)
    plsc.subcore_barrier()

    base = wid * EPW

    def body(j, _):
        off = base + j * CH
        pltpu.sync_copy(src_hbm.at[pl.ds(off, CH)], src_v)
        pltpu.sync_copy(dst_hbm.at[pl.ds(off, CH)], dst_v)
        pltpu.async_copy(hs_hbm.at[src_v], rows_v, sem).wait()
        pltpu.sync_copy(rows_v, acc_sh.at[dst_v], add=True)
        return 0
    lax.fori_loop(0, NCH, body, 0)
    plsc.subcore_barrier()

    def wback(k, _):
        r = s * RPT + k * ZR
        pltpu.sync_copy(acc_sh.at[pl.ds(r, ZR)], out_hbm.at[c, pl.ds(r, ZR)])
        return 0
    lax.fori_loop(0, RPT // ZR, wback, 0)


DDEG = 16


@functools.partial(
    pl.kernel,
    mesh=_sc_mesh,
    out_type=jax.ShapeDtypeStruct((NC, N, DDEG), jnp.float32),
    scratch_types=[
        pltpu.VMEM((CH,), jnp.int32),
        pltpu.VMEM((CH, DDEG), jnp.float32),
        pltpu.VMEM((ZR, DDEG), jnp.float32),
        pltpu.VMEM_SHARED((N, DDEG), jnp.float32),
    ],
)
def _in_degree(dst_hbm, out_hbm, dst_v, ones_v, zer_v, acc_sh):
    c = lax.axis_index("c")
    s = lax.axis_index("s")
    wid = c * NS + s

    def fill(i, _):
        zer_v[i] = jnp.zeros((16,), jnp.float32)
        return 0
    lax.fori_loop(0, ZR, fill, 0)

    def fill1(i, _):
        ones_v[i] = jnp.ones((16,), jnp.float32)
        return 0
    lax.fori_loop(0, CH, fill1, 0)

    def zcopy(k, _):
        pltpu.sync_copy(zer_v, acc_sh.at[pl.ds(s * RPT + k * ZR, ZR)])
        return 0
    lax.fori_loop(0, RPT // ZR, zcopy, 0)
    plsc.subcore_barrier()

    base = wid * EPW

    def body(j, _):
        pltpu.sync_copy(dst_hbm.at[pl.ds(base + j * CH, CH)], dst_v)
        pltpu.sync_copy(ones_v, acc_sh.at[dst_v], add=True)
        return 0
    lax.fori_loop(0, NCH, body, 0)
    plsc.subcore_barrier()

    def wback(k, _):
        r = s * RPT + k * ZR
        pltpu.sync_copy(acc_sh.at[pl.ds(r, ZR)], out_hbm.at[c, pl.ds(r, ZR)])
        return 0
    lax.fori_loop(0, RPT // ZR, wback, 0)


RB = 1000
GRID = N // RB


def _tc0_body(x_ref, w_ref, indeg_ref, hs_ref):
    dinv = lax.rsqrt(indeg_ref[...] + 1.0)
    hs_ref[...] = jnp.dot(x_ref[...], w_ref[...],
                          preferred_element_type=jnp.float32) * dinv


_tc0 = pl.pallas_call(
    _tc0_body,
    grid=(GRID,),
    in_specs=[
        pl.BlockSpec((RB, D_IN), lambda i: (i, 0)),
        pl.BlockSpec((D_IN, DH), lambda i: (0, 0)),
        pl.BlockSpec((RB, 1), lambda i: (i, 0)),
    ],
    out_specs=pl.BlockSpec((RB, DH), lambda i: (i, 0)),
    out_shape=jax.ShapeDtypeStruct((N, DH), jnp.float32),
)


def _mid_body(agg_ref, hs_ref, indeg_ref, b_ref, w_ref, out_ref):
    dinv = lax.rsqrt(indeg_ref[...] + 1.0)
    t = (agg_ref[0] + agg_ref[1] + hs_ref[...]) * dinv + b_ref[...]
    t = jnp.maximum(t, 0.0)
    out_ref[...] = jnp.dot(t, w_ref[...],
                           preferred_element_type=jnp.float32) * dinv


_tc_mid = pl.pallas_call(
    _mid_body,
    grid=(GRID,),
    in_specs=[
        pl.BlockSpec((NC, RB, DH), lambda i: (0, i, 0)),
        pl.BlockSpec((RB, DH), lambda i: (i, 0)),
        pl.BlockSpec((RB, 1), lambda i: (i, 0)),
        pl.BlockSpec((1, DH), lambda i: (0, 0)),
        pl.BlockSpec((DH, DH), lambda i: (0, 0)),
    ],
    out_specs=pl.BlockSpec((RB, DH), lambda i: (i, 0)),
    out_shape=jax.ShapeDtypeStruct((N, DH), jnp.float32),
)


def _fin_body(agg_ref, hs_ref, indeg_ref, b_ref, w_ref, nb_ref,
              hfin_ref, pool_ref):
    i = pl.program_id(0)
    dinv = lax.rsqrt(indeg_ref[...] + 1.0)
    t = (agg_ref[0] + agg_ref[1] + hs_ref[...]) * dinv + b_ref[...]
    t = jnp.maximum(t, 0.0)
    hf = jnp.dot(t, w_ref[...], preferred_element_type=jnp.float32)
    hfin_ref[...] = hf
    gids = lax.broadcasted_iota(jnp.int32, (1, NG), 1)
    oh = (nb_ref[...] == gids).astype(jnp.float32)
    contrib = lax.dot_general(oh, hf, (((0,), (0,)), ((), ())),
                              preferred_element_type=jnp.float32)

    @pl.when(i == 0)
    def _():
        pool_ref[...] = contrib

    @pl.when(i > 0)
    def _():
        pool_ref[...] += contrib


_tc_fin = pl.pallas_call(
    _fin_body,
    grid=(GRID,),
    in_specs=[
        pl.BlockSpec((NC, RB, DH), lambda i: (0, i, 0)),
        pl.BlockSpec((RB, DH), lambda i: (i, 0)),
        pl.BlockSpec((RB, 1), lambda i: (i, 0)),
        pl.BlockSpec((1, DH), lambda i: (0, 0)),
        pl.BlockSpec((DH, DH), lambda i: (0, 0)),
        pl.BlockSpec((RB, 1), lambda i: (i, 0)),
    ],
    out_specs=[
        pl.BlockSpec((RB, DH), lambda i: (i, 0)),
        pl.BlockSpec((NG, DH), lambda i: (0, 0)),
    ],
    out_shape=[
        jax.ShapeDtypeStruct((N, DH), jnp.float32),
        jax.ShapeDtypeStruct((NG, DH), jnp.float32),
    ],
)


def _tail_body(hfin_ref, nb_ref, pool_ref, gw_ref, aw_ref, out_ref):
    pt = jnp.dot(pool_ref[...], gw_ref[...],
                 preferred_element_type=jnp.float32)
    gids = lax.broadcasted_iota(jnp.int32, (1, NG), 1)
    oh = (nb_ref[...] == gids).astype(jnp.float32)
    rep = jnp.dot(oh, pt, preferred_element_type=jnp.float32)
    h = jnp.maximum(hfin_ref[...], 0.0)
    r = jnp.maximum(rep, 0.0)
    a1 = aw_ref[:DH, :]
    a2 = aw_ref[DH:, :]
    z = (jnp.dot(h, a1, preferred_element_type=jnp.float32)
         + jnp.dot(r, a2, preferred_element_type=jnp.float32))
    out_ref[...] = jnp.tanh(z)


_tc_tail = pl.pallas_call(
    _tail_body,
    grid=(GRID,),
    in_specs=[
        pl.BlockSpec((RB, DH), lambda i: (i, 0)),
        pl.BlockSpec((RB, 1), lambda i: (i, 0)),
        pl.BlockSpec((NG, DH), lambda i: (0, 0)),
        pl.BlockSpec((DH, DH), lambda i: (0, 0)),
        pl.BlockSpec((2 * DH, 1), lambda i: (0, 0)),
    ],
    out_specs=pl.BlockSpec((RB, 1), lambda i: (i, 0)),
    out_shape=jax.ShapeDtypeStruct((N, 1), jnp.float32),
)


def kernel(x, edge_index, nb_batch, W0, b0, Wh, bh, node_W, grph_W, aggr_W):
    src = edge_index[0]
    dst = edge_index[1]
    nb2 = nb_batch.reshape(N, 1)

    degp = _in_degree(dst)
    indeg = degp[0, :, :1] + degp[1, :, :1]

    hs = _tc0(x, W0, indeg)
    biases = [b0] + [bh[j] for j in range(8)]
    for layer in range(8):
        agg = _edge_prop(hs, src, dst)
        hs = _tc_mid(agg, hs, indeg, biases[layer].reshape(1, DH), Wh[layer])
    agg = _edge_prop(hs, src, dst)
    hfin, pool = _tc_fin(agg, hs, indeg, biases[8].reshape(1, DH), node_W, nb2)
    return _tc_tail(hfin, nb2, pool, grph_W, aggr_W)


# speedup vs baseline: 10.2606x; 10.2606x over previous
import functools

import jax
import jax.numpy as jnp
from jax import lax
from jax.experimental import pallas as pl
from jax.experimental.pallas import tpu as pltpu
from jax.experimental.pallas import tpu_sc as plsc

N = 10000
E = 320000
D_IN = 128
DH = 64
NG = 64

NC = 2
NS = 16
NW = NC * NS
EPW = E // NW
CH = 80
NCH = EPW // CH
NP = 10240
RPT = NP // NS
ZR = 128

_sc_mesh = plsc.VectorSubcoreMesh(core_axis_name="c", subcore_axis_name="s")


@functools.partial(
    pl.kernel,
    mesh=_sc_mesh,
    out_type=jax.ShapeDtypeStruct((NC, NP, DH), jnp.float32),
    scratch_types=[
        pltpu.VMEM((CH,), jnp.int32),
        pltpu.VMEM((CH,), jnp.int32),
        pltpu.VMEM((CH, DH), jnp.float32),
        pltpu.VMEM((ZR, DH), jnp.float32),
        pltpu.VMEM_SHARED((NP, DH), jnp.float32),
        pltpu.SemaphoreType.DMA,
    ],
    compiler_params=pltpu.CompilerParams(use_tc_tiling_on_sc=False),
)
def _edge_prop(hs_hbm, src_hbm, dst_hbm, out_hbm, src_v, dst_v, rows_v,
               zer_v, acc_sh, sem):
    c = lax.axis_index("c")
    s = lax.axis_index("s")
    wid = c * NS + s

    def zfill(i, _):
        def zfill_lane(j, _):
            zer_v[i, pl.ds(j * 16, 16)] = jnp.zeros((16,), jnp.float32)
            return 0
        return lax.fori_loop(0, DH // 16, zfill_lane, 0)
    lax.fori_loop(0, ZR, zfill, 0)

    def zcopy(k, _):
        pltpu.sync_copy(zer_v, acc_sh.at[pl.ds(s * RPT + k * ZR, ZR)])
        return 0
    lax.fori_loop(0, RPT // ZR, zcopy, 0)
    plsc.subcore_barrier()

    base = wid * EPW

    def body(j, _):
        off = base + j * CH
        pltpu.sync_copy(src_hbm.at[pl.ds(off, CH)], src_v)
        pltpu.sync_copy(dst_hbm.at[pl.ds(off, CH)], dst_v)
        pltpu.async_copy(hs_hbm.at[src_v], rows_v, sem).wait()
        pltpu.sync_copy(rows_v, acc_sh.at[dst_v], add=True)
        return 0
    lax.fori_loop(0, NCH, body, 0)
    plsc.subcore_barrier()

    def wback(k, _):
        r = s * RPT + k * ZR
        pltpu.sync_copy(acc_sh.at[pl.ds(r, ZR)], out_hbm.at[c, pl.ds(r, ZR)])
        return 0
    lax.fori_loop(0, RPT // ZR, wback, 0)


DDEG = 16


@functools.partial(
    pl.kernel,
    mesh=_sc_mesh,
    out_type=jax.ShapeDtypeStruct((NC, NP, DDEG), jnp.float32),
    scratch_types=[
        pltpu.VMEM((CH,), jnp.int32),
        pltpu.VMEM((CH, DDEG), jnp.float32),
        pltpu.VMEM((ZR, DDEG), jnp.float32),
        pltpu.VMEM_SHARED((NP, DDEG), jnp.float32),
    ],
    compiler_params=pltpu.CompilerParams(use_tc_tiling_on_sc=False),
)
def _in_degree(dst_hbm, out_hbm, dst_v, ones_v, zer_v, acc_sh):
    c = lax.axis_index("c")
    s = lax.axis_index("s")
    wid = c * NS + s

    def fill(i, _):
        zer_v[i] = jnp.zeros((16,), jnp.float32)
        return 0
    lax.fori_loop(0, ZR, fill, 0)

    def fill1(i, _):
        ones_v[i] = jnp.ones((16,), jnp.float32)
        return 0
    lax.fori_loop(0, CH, fill1, 0)

    def zcopy(k, _):
        pltpu.sync_copy(zer_v, acc_sh.at[pl.ds(s * RPT + k * ZR, ZR)])
        return 0
    lax.fori_loop(0, RPT // ZR, zcopy, 0)
    plsc.subcore_barrier()

    base = wid * EPW

    def body(j, _):
        pltpu.sync_copy(dst_hbm.at[pl.ds(base + j * CH, CH)], dst_v)
        pltpu.sync_copy(ones_v, acc_sh.at[dst_v], add=True)
        return 0
    lax.fori_loop(0, NCH, body, 0)
    plsc.subcore_barrier()

    def wback(k, _):
        r = s * RPT + k * ZR
        pltpu.sync_copy(acc_sh.at[pl.ds(r, ZR)], out_hbm.at[c, pl.ds(r, ZR)])
        return 0
    lax.fori_loop(0, RPT // ZR, wback, 0)


RB = 1000
GRID = N // RB


def _tc0_body(x_ref, w_ref, indeg_ref, hs_ref):
    dinv = lax.rsqrt(indeg_ref[...] + 1.0)
    hs_ref[...] = jnp.dot(x_ref[...], w_ref[...],
                          preferred_element_type=jnp.float32) * dinv


_tc0 = pl.pallas_call(
    _tc0_body,
    grid=(GRID,),
    in_specs=[
        pl.BlockSpec((RB, D_IN), lambda i: (i, 0)),
        pl.BlockSpec((D_IN, DH), lambda i: (0, 0)),
        pl.BlockSpec((RB, 1), lambda i: (i, 0)),
    ],
    out_specs=pl.BlockSpec((RB, DH), lambda i: (i, 0)),
    out_shape=jax.ShapeDtypeStruct((N, DH), jnp.float32),
)


def _mid_body(agg_ref, hs_ref, indeg_ref, b_ref, w_ref, out_ref):
    dinv = lax.rsqrt(indeg_ref[...] + 1.0)
    t = (agg_ref[0] + agg_ref[1] + hs_ref[...]) * dinv + b_ref[...]
    t = jnp.maximum(t, 0.0)
    out_ref[...] = jnp.dot(t, w_ref[...],
                           preferred_element_type=jnp.float32) * dinv


_tc_mid = pl.pallas_call(
    _mid_body,
    grid=(GRID,),
    in_specs=[
        pl.BlockSpec((NC, RB, DH), lambda i: (0, i, 0)),
        pl.BlockSpec((RB, DH), lambda i: (i, 0)),
        pl.BlockSpec((RB, 1), lambda i: (i, 0)),
        pl.BlockSpec((1, DH), lambda i: (0, 0)),
        pl.BlockSpec((DH, DH), lambda i: (0, 0)),
    ],
    out_specs=pl.BlockSpec((RB, DH), lambda i: (i, 0)),
    out_shape=jax.ShapeDtypeStruct((N, DH), jnp.float32),
)


def _fin_body(agg_ref, hs_ref, indeg_ref, b_ref, w_ref, nb_ref,
              hfin_ref, pool_ref):
    i = pl.program_id(0)
    dinv = lax.rsqrt(indeg_ref[...] + 1.0)
    t = (agg_ref[0] + agg_ref[1] + hs_ref[...]) * dinv + b_ref[...]
    t = jnp.maximum(t, 0.0)
    hf = jnp.dot(t, w_ref[...], preferred_element_type=jnp.float32)
    hfin_ref[...] = hf
    gids = lax.broadcasted_iota(jnp.int32, (1, NG), 1)
    oh = (nb_ref[...] == gids).astype(jnp.float32)
    contrib = lax.dot_general(oh, hf, (((0,), (0,)), ((), ())),
                              preferred_element_type=jnp.float32)

    @pl.when(i == 0)
    def _():
        pool_ref[...] = contrib

    @pl.when(i > 0)
    def _():
        pool_ref[...] += contrib


_tc_fin = pl.pallas_call(
    _fin_body,
    grid=(GRID,),
    in_specs=[
        pl.BlockSpec((NC, RB, DH), lambda i: (0, i, 0)),
        pl.BlockSpec((RB, DH), lambda i: (i, 0)),
        pl.BlockSpec((RB, 1), lambda i: (i, 0)),
        pl.BlockSpec((1, DH), lambda i: (0, 0)),
        pl.BlockSpec((DH, DH), lambda i: (0, 0)),
        pl.BlockSpec((RB, 1), lambda i: (i, 0)),
    ],
    out_specs=[
        pl.BlockSpec((RB, DH), lambda i: (i, 0)),
        pl.BlockSpec((NG, DH), lambda i: (0, 0)),
    ],
    out_shape=[
        jax.ShapeDtypeStruct((N, DH), jnp.float32),
        jax.ShapeDtypeStruct((NG, DH), jnp.float32),
    ],
)


def _tail_body(hfin_ref, nb_ref, pool_ref, gw_ref, aw_ref, out_ref):
    pt = jnp.dot(pool_ref[...], gw_ref[...],
                 preferred_element_type=jnp.float32)
    gids = lax.broadcasted_iota(jnp.int32, (1, NG), 1)
    oh = (nb_ref[...] == gids).astype(jnp.float32)
    rep = jnp.dot(oh, pt, preferred_element_type=jnp.float32)
    h = jnp.maximum(hfin_ref[...], 0.0)
    r = jnp.maximum(rep, 0.0)
    a1 = aw_ref[:DH, :]
    a2 = aw_ref[DH:, :]
    z = (jnp.dot(h, a1, preferred_element_type=jnp.float32)
         + jnp.dot(r, a2, preferred_element_type=jnp.float32))
    out_ref[...] = jnp.tanh(z)


_tc_tail = pl.pallas_call(
    _tail_body,
    grid=(GRID,),
    in_specs=[
        pl.BlockSpec((RB, DH), lambda i: (i, 0)),
        pl.BlockSpec((RB, 1), lambda i: (i, 0)),
        pl.BlockSpec((NG, DH), lambda i: (0, 0)),
        pl.BlockSpec((DH, DH), lambda i: (0, 0)),
        pl.BlockSpec((2 * DH, 1), lambda i: (0, 0)),
    ],
    out_specs=pl.BlockSpec((RB, 1), lambda i: (i, 0)),
    out_shape=jax.ShapeDtypeStruct((N, 1), jnp.float32),
)


def kernel(x, edge_index, nb_batch, W0, b0, Wh, bh, node_W, grph_W, aggr_W):
    src = edge_index[0]
    dst = edge_index[1]
    nb2 = nb_batch.reshape(N, 1)

    degp = _in_degree(dst)
    indeg = degp[0, :N, :1] + degp[1, :N, :1]

    hs = _tc0(x, W0, indeg)
    biases = [b0] + [bh[j] for j in range(8)]
    for layer in range(8):
        agg = _edge_prop(hs, src, dst)
        hs = _tc_mid(agg, hs, indeg, biases[layer].reshape(1, DH), Wh[layer])
    agg = _edge_prop(hs, src, dst)
    hfin, pool = _tc_fin(agg, hs, indeg, biases[8].reshape(1, DH), node_W, nb2)
    return _tc_tail(hfin, nb2, pool, grph_W, aggr_W)


# speedup vs baseline: 31.3806x; 3.0584x over previous
import functools

import jax
import jax.numpy as jnp
from jax import lax
from jax.experimental import pallas as pl
from jax.experimental.pallas import tpu as pltpu
from jax.experimental.pallas import tpu_sc as plsc

N = 10000
E = 320000
D_IN = 128
DH = 64
NG = 64

NC = 2
NS = 16
NW = NC * NS
EPW = E // NW
CH = 80
NCH = EPW // CH
NP = 10240
RPT = NP // NS
ZR = 128

_sc_mesh = plsc.VectorSubcoreMesh(core_axis_name="c", subcore_axis_name="s")


NB = 5
NGRP = NCH // NB


@functools.partial(
    pl.kernel,
    mesh=_sc_mesh,
    out_type=jax.ShapeDtypeStruct((NC, NP, DH), jnp.float32),
    scratch_types=[
        pltpu.VMEM((NCH, CH), jnp.int32),
        pltpu.VMEM((NCH, CH), jnp.int32),
        pltpu.VMEM((2 * NB, CH, DH), jnp.float32),
        pltpu.VMEM((ZR, DH), jnp.float32),
        pltpu.VMEM_SHARED((NP, DH), jnp.float32),
        pltpu.SemaphoreType.DMA,
        pltpu.SemaphoreType.DMA,
    ],
    compiler_params=pltpu.CompilerParams(use_tc_tiling_on_sc=False),
)
def _edge_prop(hs_hbm, src_hbm, dst_hbm, out_hbm, src_all, dst_all, rows_v,
               zer_v, acc_sh, sem_g, sem_s):
    c = lax.axis_index("c")
    s = lax.axis_index("s")
    wid = c * NS + s

    g1 = pltpu.async_copy(src_hbm.at[wid], src_all, sem_g)
    g2 = pltpu.async_copy(dst_hbm.at[wid], dst_all, sem_g)

    def zfill(i, _):
        def zfill_lane(j, _):
            zer_v[i, pl.ds(j * 16, 16)] = jnp.zeros((16,), jnp.float32)
            return 0
        return lax.fori_loop(0, DH // 16, zfill_lane, 0)
    lax.fori_loop(0, ZR, zfill, 0)

    def zcopy(k, _):
        pltpu.sync_copy(zer_v, acc_sh.at[pl.ds(s * RPT + k * ZR, ZR)])
        return 0
    lax.fori_loop(0, RPT // ZR, zcopy, 0)
    g1.wait()
    g2.wait()
    plsc.subcore_barrier()

    for b in range(NB):
        pltpu.async_copy(hs_hbm.at[src_all.at[b]], rows_v.at[b], sem_g)

    def grp(t, _):
        pb = (t % 2) * NB
        for b in range(NB):
            pltpu.make_async_copy(hs_hbm.at[src_all.at[t * NB + b]],
                                  rows_v.at[pb + b], sem_g).wait()
        for b in range(NB):
            pltpu.async_copy(rows_v.at[pb + b],
                             acc_sh.at[dst_all.at[t * NB + b]], sem_s,
                             add=True)

        @pl.when(t < NGRP - 1)
        def _():
            qb = ((t + 1) % 2) * NB
            for b in range(NB):
                pltpu.async_copy(hs_hbm.at[src_all.at[(t + 1) * NB + b]],
                                 rows_v.at[qb + b], sem_g)

        for b in range(NB):
            pltpu.make_async_copy(rows_v.at[pb + b],
                                  acc_sh.at[dst_all.at[t * NB + b]],
                                  sem_s).wait()
        return 0
    lax.fori_loop(0, NGRP, grp, 0)
    plsc.subcore_barrier()

    def wback(k, _):
        r = s * RPT + k * ZR
        pltpu.sync_copy(acc_sh.at[pl.ds(r, ZR)], out_hbm.at[c, pl.ds(r, ZR)])
        return 0
    lax.fori_loop(0, RPT // ZR, wback, 0)


DDEG = 16


@functools.partial(
    pl.kernel,
    mesh=_sc_mesh,
    out_type=jax.ShapeDtypeStruct((NC, NP, DDEG), jnp.float32),
    scratch_types=[
        pltpu.VMEM((NCH, CH), jnp.int32),
        pltpu.VMEM((CH, DDEG), jnp.float32),
        pltpu.VMEM((ZR, DDEG), jnp.float32),
        pltpu.VMEM_SHARED((NP, DDEG), jnp.float32),
        pltpu.SemaphoreType.DMA,
        pltpu.SemaphoreType.DMA,
    ],
    compiler_params=pltpu.CompilerParams(use_tc_tiling_on_sc=False),
)
def _in_degree(dst_hbm, out_hbm, dst_all, ones_v, zer_v, acc_sh,
               sem_g, sem_s):
    c = lax.axis_index("c")
    s = lax.axis_index("s")
    wid = c * NS + s

    g1 = pltpu.async_copy(dst_hbm.at[wid], dst_all, sem_g)

    def fill(i, _):
        zer_v[i] = jnp.zeros((16,), jnp.float32)
        return 0
    lax.fori_loop(0, ZR, fill, 0)

    def fill1(i, _):
        ones_v[i] = jnp.ones((16,), jnp.float32)
        return 0
    lax.fori_loop(0, CH, fill1, 0)

    def zcopy(k, _):
        pltpu.sync_copy(zer_v, acc_sh.at[pl.ds(s * RPT + k * ZR, ZR)])
        return 0
    lax.fori_loop(0, RPT // ZR, zcopy, 0)
    g1.wait()
    plsc.subcore_barrier()

    def grp(t, _):
        for b in range(NB):
            pltpu.async_copy(ones_v, acc_sh.at[dst_all.at[t * NB + b]],
                             sem_s, add=True)
        for b in range(NB):
            pltpu.make_async_copy(ones_v, acc_sh.at[dst_all.at[t * NB + b]],
                                  sem_s).wait()
        return 0
    lax.fori_loop(0, NGRP, grp, 0)
    plsc.subcore_barrier()

    def wback(k, _):
        r = s * RPT + k * ZR
        pltpu.sync_copy(acc_sh.at[pl.ds(r, ZR)], out_hbm.at[c, pl.ds(r, ZR)])
        return 0
    lax.fori_loop(0, RPT // ZR, wback, 0)


RB = 1000
GRID = N // RB


def _tc0_body(x_ref, w_ref, indeg_ref, hs_ref):
    dinv = lax.rsqrt(indeg_ref[...] + 1.0)
    hs_ref[...] = jnp.dot(x_ref[...], w_ref[...],
                          preferred_element_type=jnp.float32) * dinv


_tc0 = pl.pallas_call(
    _tc0_body,
    grid=(GRID,),
    in_specs=[
        pl.BlockSpec((RB, D_IN), lambda i: (i, 0)),
        pl.BlockSpec((D_IN, DH), lambda i: (0, 0)),
        pl.BlockSpec((RB, 1), lambda i: (i, 0)),
    ],
    out_specs=pl.BlockSpec((RB, DH), lambda i: (i, 0)),
    out_shape=jax.ShapeDtypeStruct((N, DH), jnp.float32),
)


def _mid_body(agg_ref, hs_ref, indeg_ref, b_ref, w_ref, out_ref):
    dinv = lax.rsqrt(indeg_ref[...] + 1.0)
    t = (agg_ref[0] + agg_ref[1] + hs_ref[...]) * dinv + b_ref[...]
    t = jnp.maximum(t, 0.0)
    out_ref[...] = jnp.dot(t, w_ref[...],
                           preferred_element_type=jnp.float32) * dinv


_tc_mid = pl.pallas_call(
    _mid_body,
    grid=(GRID,),
    in_specs=[
        pl.BlockSpec((NC, RB, DH), lambda i: (0, i, 0)),
        pl.BlockSpec((RB, DH), lambda i: (i, 0)),
        pl.BlockSpec((RB, 1), lambda i: (i, 0)),
        pl.BlockSpec((1, DH), lambda i: (0, 0)),
        pl.BlockSpec((DH, DH), lambda i: (0, 0)),
    ],
    out_specs=pl.BlockSpec((RB, DH), lambda i: (i, 0)),
    out_shape=jax.ShapeDtypeStruct((N, DH), jnp.float32),
)


def _fin_body(agg_ref, hs_ref, indeg_ref, b_ref, w_ref, nb_ref,
              hfin_ref, pool_ref):
    i = pl.program_id(0)
    dinv = lax.rsqrt(indeg_ref[...] + 1.0)
    t = (agg_ref[0] + agg_ref[1] + hs_ref[...]) * dinv + b_ref[...]
    t = jnp.maximum(t, 0.0)
    hf = jnp.dot(t, w_ref[...], preferred_element_type=jnp.float32)
    hfin_ref[...] = hf
    gids = lax.broadcasted_iota(jnp.int32, (1, NG), 1)
    oh = (nb_ref[...] == gids).astype(jnp.float32)
    contrib = lax.dot_general(oh, hf, (((0,), (0,)), ((), ())),
                              preferred_element_type=jnp.float32)

    @pl.when(i == 0)
    def _():
        pool_ref[...] = contrib

    @pl.when(i > 0)
    def _():
        pool_ref[...] += contrib


_tc_fin = pl.pallas_call(
    _fin_body,
    grid=(GRID,),
    in_specs=[
        pl.BlockSpec((NC, RB, DH), lambda i: (0, i, 0)),
        pl.BlockSpec((RB, DH), lambda i: (i, 0)),
        pl.BlockSpec((RB, 1), lambda i: (i, 0)),
        pl.BlockSpec((1, DH), lambda i: (0, 0)),
        pl.BlockSpec((DH, DH), lambda i: (0, 0)),
        pl.BlockSpec((RB, 1), lambda i: (i, 0)),
    ],
    out_specs=[
        pl.BlockSpec((RB, DH), lambda i: (i, 0)),
        pl.BlockSpec((NG, DH), lambda i: (0, 0)),
    ],
    out_shape=[
        jax.ShapeDtypeStruct((N, DH), jnp.float32),
        jax.ShapeDtypeStruct((NG, DH), jnp.float32),
    ],
)


def _tail_body(hfin_ref, nb_ref, pool_ref, gw_ref, aw_ref, out_ref):
    pt = jnp.dot(pool_ref[...], gw_ref[...],
                 preferred_element_type=jnp.float32)
    gids = lax.broadcasted_iota(jnp.int32, (1, NG), 1)
    oh = (nb_ref[...] == gids).astype(jnp.float32)
    rep = jnp.dot(oh, pt, preferred_element_type=jnp.float32)
    h = jnp.maximum(hfin_ref[...], 0.0)
    r = jnp.maximum(rep, 0.0)
    a1 = aw_ref[:DH, :]
    a2 = aw_ref[DH:, :]
    z = (jnp.dot(h, a1, preferred_element_type=jnp.float32)
         + jnp.dot(r, a2, preferred_element_type=jnp.float32))
    out_ref[...] = jnp.tanh(z)


_tc_tail = pl.pallas_call(
    _tail_body,
    grid=(GRID,),
    in_specs=[
        pl.BlockSpec((RB, DH), lambda i: (i, 0)),
        pl.BlockSpec((RB, 1), lambda i: (i, 0)),
        pl.BlockSpec((NG, DH), lambda i: (0, 0)),
        pl.BlockSpec((DH, DH), lambda i: (0, 0)),
        pl.BlockSpec((2 * DH, 1), lambda i: (0, 0)),
    ],
    out_specs=pl.BlockSpec((RB, 1), lambda i: (i, 0)),
    out_shape=jax.ShapeDtypeStruct((N, 1), jnp.float32),
)


def kernel(x, edge_index, nb_batch, W0, b0, Wh, bh, node_W, grph_W, aggr_W):
    src3 = edge_index[0].reshape(NW, NCH, CH)
    dst3 = edge_index[1].reshape(NW, NCH, CH)
    nb2 = nb_batch.reshape(N, 1)

    degp = _in_degree(dst3)
    indeg = degp[0, :N, :1] + degp[1, :N, :1]

    hs = _tc0(x, W0, indeg)
    biases = [b0] + [bh[j] for j in range(8)]
    for layer in range(8):
        agg = _edge_prop(hs, src3, dst3)
        hs = _tc_mid(agg, hs, indeg, biases[layer].reshape(1, DH), Wh[layer])
    agg = _edge_prop(hs, src3, dst3)
    hfin, pool = _tc_fin(agg, hs, indeg, biases[8].reshape(1, DH), node_W, nb2)
    return _tc_tail(hfin, nb2, pool, grph_W, aggr_W)


# speedup vs baseline: 40.8804x; 1.3027x over previous
import functools

import jax
import jax.numpy as jnp
from jax import lax
from jax.experimental import pallas as pl
from jax.experimental.pallas import tpu as pltpu
from jax.experimental.pallas import tpu_sc as plsc

N = 10000
E = 320000
D_IN = 128
DH = 64
NG = 64

NC = 2
NS = 16
NW = NC * NS
EPW = E // NW
CH = 80
NCH = EPW // CH
NP = 10240
RPT = NP // NS
ZR = 128

_sc_mesh = plsc.VectorSubcoreMesh(core_axis_name="c", subcore_axis_name="s")


NB = 5
NGRP = NCH // NB


@functools.partial(
    pl.kernel,
    mesh=_sc_mesh,
    out_type=jax.ShapeDtypeStruct((NC, NP, DH), jnp.float32),
    scratch_types=[
        pltpu.VMEM((NCH, CH), jnp.int32),
        pltpu.VMEM((NCH, CH), jnp.int32),
        pltpu.VMEM((2 * NB, CH, DH), jnp.float32),
        pltpu.VMEM((ZR, DH), jnp.float32),
        pltpu.VMEM_SHARED((NP, DH), jnp.float32),
        pltpu.SemaphoreType.DMA,
        pltpu.SemaphoreType.DMA,
    ],
    compiler_params=pltpu.CompilerParams(use_tc_tiling_on_sc=False),
)
def _edge_prop(hs_hbm, src_hbm, dst_hbm, out_hbm, src_all, dst_all, rows_v,
               zer_v, acc_sh, sem_g, sem_s):
    c = lax.axis_index("c")
    s = lax.axis_index("s")
    wid = c * NS + s

    g1 = pltpu.async_copy(src_hbm.at[wid], src_all, sem_g)
    g2 = pltpu.async_copy(dst_hbm.at[wid], dst_all, sem_g)

    def zfill(i, _):
        def zfill_lane(j, _):
            zer_v[i, pl.ds(j * 16, 16)] = jnp.zeros((16,), jnp.float32)
            return 0
        return lax.fori_loop(0, DH // 16, zfill_lane, 0)
    lax.fori_loop(0, ZR, zfill, 0)

    def zcopy(k, _):
        pltpu.sync_copy(zer_v, acc_sh.at[pl.ds(s * RPT + k * ZR, ZR)])
        return 0
    lax.fori_loop(0, RPT // ZR, zcopy, 0)
    g1.wait()
    g2.wait()
    plsc.subcore_barrier()

    for b in range(NB):
        pltpu.async_copy(hs_hbm.at[src_all.at[b]], rows_v.at[b], sem_g)

    def grp(t, _):
        pb = (t % 2) * NB
        for b in range(NB):
            pltpu.make_async_copy(hs_hbm.at[src_all.at[t * NB + b]],
                                  rows_v.at[pb + b], sem_g).wait()
        for b in range(NB):
            pltpu.async_copy(rows_v.at[pb + b],
                             acc_sh.at[dst_all.at[t * NB + b]], sem_s,
                             add=True)

        @pl.when(t < NGRP - 1)
        def _():
            qb = ((t + 1) % 2) * NB
            for b in range(NB):
                pltpu.async_copy(hs_hbm.at[src_all.at[(t + 1) * NB + b]],
                                 rows_v.at[qb + b], sem_g)

        for b in range(NB):
            pltpu.make_async_copy(rows_v.at[pb + b],
                                  acc_sh.at[dst_all.at[t * NB + b]],
                                  sem_s).wait()
        return 0
    lax.fori_loop(0, NGRP, grp, 0)
    plsc.subcore_barrier()

    def wback(k, _):
        r = s * RPT + k * ZR
        pltpu.sync_copy(acc_sh.at[pl.ds(r, ZR)], out_hbm.at[c, pl.ds(r, ZR)])
        return 0
    lax.fori_loop(0, RPT // ZR, wback, 0)


DDEG = 16


@functools.partial(
    pl.kernel,
    mesh=_sc_mesh,
    out_type=jax.ShapeDtypeStruct((NC, NP, DH), jnp.float32),
    scratch_types=[
        pltpu.VMEM((NCH, CH), jnp.int32),
        pltpu.VMEM((CH, DDEG), jnp.float32),
        pltpu.VMEM((ZR, DDEG), jnp.float32),
        pltpu.VMEM((ZR, DDEG), jnp.float32),
        pltpu.VMEM((ZR, DH), jnp.float32),
        pltpu.VMEM_SHARED((NP, DDEG), jnp.float32),
        pltpu.SemaphoreType.DMA,
        pltpu.SemaphoreType.DMA,
    ],
    compiler_params=pltpu.CompilerParams(use_tc_tiling_on_sc=False),
)
def _in_degree(dst_hbm, out_hbm, dst_all, ones_v, zer_v, a16_v, a64_v, acc_sh,
               sem_g, sem_s):
    c = lax.axis_index("c")
    s = lax.axis_index("s")
    wid = c * NS + s

    g1 = pltpu.async_copy(dst_hbm.at[wid], dst_all, sem_g)

    def fill(i, _):
        zer_v[i] = jnp.zeros((16,), jnp.float32)
        return 0
    lax.fori_loop(0, ZR, fill, 0)

    def fill1(i, _):
        ones_v[i] = jnp.ones((16,), jnp.float32)
        return 0
    lax.fori_loop(0, CH, fill1, 0)

    def zcopy(k, _):
        pltpu.sync_copy(zer_v, acc_sh.at[pl.ds(s * RPT + k * ZR, ZR)])
        return 0
    lax.fori_loop(0, RPT // ZR, zcopy, 0)
    g1.wait()
    plsc.subcore_barrier()

    def grp(t, _):
        for b in range(NB):
            pltpu.async_copy(ones_v, acc_sh.at[dst_all.at[t * NB + b]],
                             sem_s, add=True)
        for b in range(NB):
            pltpu.make_async_copy(ones_v, acc_sh.at[dst_all.at[t * NB + b]],
                                  sem_s).wait()
        return 0
    lax.fori_loop(0, NGRP, grp, 0)
    plsc.subcore_barrier()

    def wback(k, _):
        r = s * RPT + k * ZR
        pltpu.sync_copy(acc_sh.at[pl.ds(r, ZR)], a16_v)

        def expand(i, _):
            v = a16_v[i]

            def lanes(q, _):
                a64_v[i, pl.ds(q * DDEG, DDEG)] = v
                return 0
            return lax.fori_loop(0, DH // DDEG, lanes, 0)
        lax.fori_loop(0, ZR, expand, 0)
        pltpu.sync_copy(a64_v, out_hbm.at[c, pl.ds(r, ZR)])
        return 0
    lax.fori_loop(0, RPT // ZR, wback, 0)


NRR = N // 2
RB = 1000
GRID = NRR // RB


def _tc0_body(x_ref, w_ref, degp_ref, hs_ref, dv_ref):
    dv = lax.rsqrt(degp_ref[0] + degp_ref[1] + 1.0)
    dv_ref[...] = dv
    hs_ref[...] = jnp.dot(x_ref[...], w_ref[...],
                          preferred_element_type=jnp.float32) * dv


_tc0 = pl.pallas_call(
    _tc0_body,
    grid=(GRID,),
    in_specs=[
        pl.BlockSpec((RB, 2 * D_IN), lambda i: (i, 0)),
        pl.BlockSpec((2 * D_IN, 2 * DH), lambda i: (0, 0)),
        pl.BlockSpec((NC, RB, 2 * DH), lambda i: (0, i, 0)),
    ],
    out_specs=[
        pl.BlockSpec((RB, 2 * DH), lambda i: (i, 0)),
        pl.BlockSpec((RB, 2 * DH), lambda i: (i, 0)),
    ],
    out_shape=[
        jax.ShapeDtypeStruct((NRR, 2 * DH), jnp.float32),
        jax.ShapeDtypeStruct((NRR, 2 * DH), jnp.float32),
    ],
)


def _mid_body(agg_ref, hs_ref, dv_ref, b_ref, w_ref, out_ref):
    dv = dv_ref[...]
    t = (agg_ref[0] + agg_ref[1] + hs_ref[...]) * dv + b_ref[...]
    t = jnp.maximum(t, 0.0)
    out_ref[...] = jnp.dot(t, w_ref[...],
                           preferred_element_type=jnp.float32) * dv


_tc_mid = pl.pallas_call(
    _mid_body,
    grid=(GRID,),
    in_specs=[
        pl.BlockSpec((NC, RB, 2 * DH), lambda i: (0, i, 0)),
        pl.BlockSpec((RB, 2 * DH), lambda i: (i, 0)),
        pl.BlockSpec((RB, 2 * DH), lambda i: (i, 0)),
        pl.BlockSpec((1, 2 * DH), lambda i: (0, 0)),
        pl.BlockSpec((2 * DH, 2 * DH), lambda i: (0, 0)),
    ],
    out_specs=pl.BlockSpec((RB, 2 * DH), lambda i: (i, 0)),
    out_shape=jax.ShapeDtypeStruct((NRR, 2 * DH), jnp.float32),
)


def _head_body(agg_ref, hs_ref, dv_ref, b_ref, w_ref, nb_ref,
               gw_ref, aw_ref, out_ref, hf_s, pool_s, pt_s):
    i = pl.program_id(0)
    gids = lax.broadcasted_iota(jnp.int32, (1, NG), 1)
    oh_a = (nb_ref[:, 0:1] == gids).astype(jnp.float32)
    oh_b = (nb_ref[:, 1:2] == gids).astype(jnp.float32)

    @pl.when(i < GRID)
    def _():
        dv = dv_ref[...]
        t = (agg_ref[0] + agg_ref[1] + hs_ref[...]) * dv + b_ref[...]
        t = jnp.maximum(t, 0.0)
        hf = jnp.dot(t, w_ref[...], preferred_element_type=jnp.float32)
        hf_s[pl.ds(i * RB, RB), :] = hf
        contrib = (
            lax.dot_general(oh_a, hf[:, :DH], (((0,), (0,)), ((), ())),
                            preferred_element_type=jnp.float32)
            + lax.dot_general(oh_b, hf[:, DH:], (((0,), (0,)), ((), ())),
                              preferred_element_type=jnp.float32))

        @pl.when(i == 0)
        def _():
            pool_s[...] = contrib

        @pl.when(i > 0)
        def _():
            pool_s[...] += contrib

    @pl.when(i >= GRID)
    def _():
        @pl.when(i == GRID)
        def _():
            pt_s[...] = jnp.dot(pool_s[...], gw_ref[...],
                                preferred_element_type=jnp.float32)
        pt = pt_s[...]
        rep_a = jnp.dot(oh_a, pt, preferred_element_type=jnp.float32)
        rep_b = jnp.dot(oh_b, pt, preferred_element_type=jnp.float32)
        hf = jnp.maximum(hf_s[pl.ds((i - GRID) * RB, RB), :], 0.0)
        a1 = aw_ref[:DH, :]
        a2 = aw_ref[DH:, :]
        za = (jnp.dot(hf[:, :DH], a1, preferred_element_type=jnp.float32)
              + jnp.dot(jnp.maximum(rep_a, 0.0), a2,
                        preferred_element_type=jnp.float32))
        zb = (jnp.dot(hf[:, DH:], a1, preferred_element_type=jnp.float32)
              + jnp.dot(jnp.maximum(rep_b, 0.0), a2,
                        preferred_element_type=jnp.float32))
        out_ref[...] = jnp.tanh(jnp.concatenate([za, zb], axis=1))


def _pmap(i):
    return jnp.where(i < GRID, i, i - GRID)


_tc_head = pl.pallas_call(
    _head_body,
    grid=(2 * GRID,),
    in_specs=[
        pl.BlockSpec((NC, RB, 2 * DH), lambda i: (0, _pmap(i), 0)),
        pl.BlockSpec((RB, 2 * DH), lambda i: (_pmap(i), 0)),
        pl.BlockSpec((RB, 2 * DH), lambda i: (_pmap(i), 0)),
        pl.BlockSpec((1, 2 * DH), lambda i: (0, 0)),
        pl.BlockSpec((2 * DH, 2 * DH), lambda i: (0, 0)),
        pl.BlockSpec((RB, 2), lambda i: (_pmap(i), 0)),
        pl.BlockSpec((DH, DH), lambda i: (0, 0)),
        pl.BlockSpec((2 * DH, 1), lambda i: (0, 0)),
    ],
    out_specs=pl.BlockSpec((RB, 2), lambda i: (_pmap(i), 0)),
    out_shape=jax.ShapeDtypeStruct((NRR, 2), jnp.float32),
    scratch_shapes=[
        pltpu.VMEM((NRR, 2 * DH), jnp.float32),
        pltpu.VMEM((NG, DH), jnp.float32),
        pltpu.VMEM((NG, DH), jnp.float32),
    ],
)


EB = 16384
EGRID = -(-E // EB)


def _split_body(ei_ref, src_ref, dst_ref):
    v = ei_ref[...]
    src_ref[...] = v[0]
    dst_ref[...] = v[1]


_tc_split = pl.pallas_call(
    _split_body,
    grid=(EGRID,),
    in_specs=[pl.BlockSpec((2, EB), lambda i: (0, i))],
    out_specs=[
        pl.BlockSpec((EB,), lambda i: (i,)),
        pl.BlockSpec((EB,), lambda i: (i,)),
    ],
    out_shape=[
        jax.ShapeDtypeStruct((E,), jnp.int32),
        jax.ShapeDtypeStruct((E,), jnp.int32),
    ],
)


def _blockdiag(w):
    z = jnp.zeros_like(w)
    return jnp.concatenate(
        [jnp.concatenate([w, z], axis=1), jnp.concatenate([z, w], axis=1)],
        axis=0)


def kernel(x, edge_index, nb_batch, W0, b0, Wh, bh, node_W, grph_W, aggr_W):
    src_lin, dst_lin = _tc_split(edge_index)
    src3 = src_lin.reshape(NW, NCH, CH)
    dst3 = dst_lin.reshape(NW, NCH, CH)
    nb_pk = nb_batch.reshape(NRR, 2)

    degp = _in_degree(dst3)
    x2 = x.reshape(NRR, 2 * D_IN)
    hs_pk, dv_pk = _tc0(x2, _blockdiag(W0),
                        degp.reshape(NC, NP // 2, 2 * DH))
    biases = [b0] + [bh[j] for j in range(8)]
    for layer in range(8):
        agg = _edge_prop(hs_pk.reshape(N, DH), src3, dst3)
        agg_pk = agg.reshape(NC, NP // 2, 2 * DH)
        hs_pk = _tc_mid(agg_pk, hs_pk, dv_pk,
                        jnp.concatenate([biases[layer]] * 2).reshape(1, 2 * DH),
                        _blockdiag(Wh[layer]))
    agg = _edge_prop(hs_pk.reshape(N, DH), src3, dst3)
    agg_pk = agg.reshape(NC, NP // 2, 2 * DH)
    out = _tc_head(agg_pk, hs_pk, dv_pk,
                   jnp.concatenate([biases[8]] * 2).reshape(1, 2 * DH),
                   _blockdiag(node_W), nb_pk, grph_W, aggr_W)
    return out.reshape(N, 1)
